# Initial kernel scaffold; baseline (speedup 1.0000x reference)
import jax
import jax.numpy as jnp
from jax.experimental import pallas as pl


def kernel(predictions):
    raise NotImplementedError("write your pallas kernel here")


# speedup vs baseline: 17.8292x; 17.8292x over previous
import jax
import jax.numpy as jnp
from jax.experimental import pallas as pl
from jax.experimental.pallas import tpu as pltpu

_CONF = 0.7
_IOU = 0.45
_MAXDET = 300
_K = 1000
_N = 20000
_ROWS = 160
_COLS = 128
_NPAD = _ROWS * _COLS

_BITS_NEG1 = -1082130432
_BITS_TWO = 1073741824


def _nms_body(fields_ref, out_ref, x1_ref, y1_ref, x2_ref, y2_ref,
              area_ref, pit_ref, yaw_ref, rol_ref):
    cx = fields_ref[0]
    cy = fields_ref[1]
    w = fields_ref[2]
    h = fields_ref[3]
    conf = fields_ref[4] * fields_ref[5]
    s0 = jnp.where(conf > _CONF, conf, -1.0)

    x1 = cx - w * 0.5
    y1 = cy - h * 0.5
    x2 = cx + w * 0.5
    y2 = cy + h * 0.5
    x1_ref[:, :] = x1
    y1_ref[:, :] = y1
    x2_ref[:, :] = x2
    y2_ref[:, :] = y2
    area_ref[:, :] = (x2 - x1) * (y2 - y1)
    pit_ref[:, :] = fields_ref[6]
    yaw_ref[:, :] = fields_ref[7]
    rol_ref[:, :] = fields_ref[8]

    s_int = jax.lax.bitcast_convert_type(s0, jnp.int32)
    kf = jnp.float32(_K)

    def bis_body(_, lh):
        lo, hi = lh
        mid = lo + (hi - lo + 1) // 2
        cnt = jnp.sum(jnp.where(s_int >= mid, 1.0, 0.0))
        ok = cnt >= kf
        return (jnp.where(ok, mid, lo), jnp.where(ok, hi, mid - 1))

    v, _ = jax.lax.fori_loop(
        0, 32, bis_body, (jnp.int32(_BITS_NEG1), jnp.int32(_BITS_TWO)))

    idx2d = (jax.lax.broadcasted_iota(jnp.int32, (_ROWS, _COLS), 0) * _COLS
             + jax.lax.broadcasted_iota(jnp.int32, (_ROWS, _COLS), 1))

    gt = s_int > v
    eq = s_int == v
    c_gt = jnp.sum(jnp.where(gt, 1.0, 0.0))

    def jbody(_, lh):
        lo, hi = lh
        mid = (lo + hi) // 2
        cnt = c_gt + jnp.sum(jnp.where(eq & (idx2d <= mid), 1.0, 0.0))
        ok = cnt >= kf
        return (jnp.where(ok, lo, mid + 1), jnp.where(ok, mid, hi))

    jcut, _ = jax.lax.fori_loop(
        0, 15, jbody, (jnp.int32(0), jnp.int32(_NPAD - 1)))

    keep = gt | (eq & (idx2d <= jcut))
    s_start = jnp.where(keep, s0, -1.0)

    oh_iota = (jax.lax.broadcasted_iota(jnp.int32, (8, _COLS), 0) * _COLS
               + jax.lax.broadcasted_iota(jnp.int32, (8, _COLS), 1))
    lane = jax.lax.broadcasted_iota(jnp.int32, (1, _COLS), 1)

    def step(t, carry):
        s, ax1, ay1, ax2, ay2, acf, apt, ayw, arl = carry
        m = jnp.max(s)
        valid = jnp.where(m > 0.0, 1.0, 0.0)
        bidx = jnp.min(jnp.where(s == m, idx2d, jnp.int32(_NPAD)))
        r = bidx // _COLS
        c = bidx % _COLS
        lm = jnp.where(lane == c, 1.0, 0.0)

        def ext(ref):
            return jnp.sum(ref[pl.ds(r, 1), :] * lm)

        bx1 = ext(x1_ref)
        by1 = ext(y1_ref)
        bx2 = ext(x2_ref)
        by2 = ext(y2_ref)
        bpt = ext(pit_ref)
        byw = ext(yaw_ref)
        brl = ext(rol_ref)
        ba = (bx2 - bx1) * (by2 - by1)

        ix1 = jnp.maximum(bx1, x1_ref[:, :])
        iy1 = jnp.maximum(by1, y1_ref[:, :])
        ix2 = jnp.minimum(bx2, x2_ref[:, :])
        iy2 = jnp.minimum(by2, y2_ref[:, :])
        inter = (jnp.clip(ix2 - ix1, 0.0) * jnp.clip(iy2 - iy1, 0.0))
        iou = inter / (ba + area_ref[:, :] - inter + 1e-9)
        supp = (iou > _IOU) | (idx2d == bidx)
        s = jnp.where(supp, -1.0, s)

        oh = jnp.where(oh_iota == t, valid, 0.0)
        return (s,
                ax1 + oh * bx1, ay1 + oh * by1,
                ax2 + oh * bx2, ay2 + oh * by2,
                acf + oh * m,
                apt + oh * bpt, ayw + oh * byw, arl + oh * brl)

    z = jnp.zeros((8, _COLS), jnp.float32)
    carry = jax.lax.fori_loop(
        0, _MAXDET, step,
        (s_start, z, z, z, z, z, z, z, z))
    _, ax1, ay1, ax2, ay2, acf, apt, ayw, arl = carry

    out_ref[0, :, :] = ax1
    out_ref[1, :, :] = ay1
    out_ref[2, :, :] = ax2
    out_ref[3, :, :] = ay2
    out_ref[4, :, :] = acf
    out_ref[5, :, :] = z
    out_ref[6, :, :] = apt
    out_ref[7, :, :] = ayw
    out_ref[8, :, :] = arl


def _nms_call(fields):
    scr = pltpu.VMEM((_ROWS, _COLS), jnp.float32)
    return pl.pallas_call(
        _nms_body,
        out_shape=jax.ShapeDtypeStruct((9, 8, _COLS), jnp.float32),
        scratch_shapes=[scr] * 8,
    )(fields)


def kernel(predictions):
    p = predictions[0]
    pt = jnp.pad(p.T, ((0, 0), (0, _NPAD - _N)))
    fields = pt.reshape(9, _ROWS, _COLS)
    out = _nms_call(fields)
    return out.reshape(9, 8 * _COLS)[:, :_MAXDET].T


# speedup vs baseline: 17.8670x; 1.0021x over previous
import jax
import jax.numpy as jnp
from jax.experimental import pallas as pl
from jax.experimental.pallas import tpu as pltpu

_CONF = 0.7
_IOU = 0.45
_MAXDET = 300
_K = 1000
_N = 20000
_ROWS = 160
_COLS = 128
_NPAD = _ROWS * _COLS

_BITS_NEG1 = -1082130432
_BITS_ONE = 1065353216


def _nms_body(fields_ref, out_ref, x1_ref, y1_ref, x2_ref, y2_ref,
              area_ref, pit_ref, yaw_ref, rol_ref):
    cx = fields_ref[0]
    cy = fields_ref[1]
    w = fields_ref[2]
    h = fields_ref[3]
    conf = fields_ref[4] * fields_ref[5]
    s0 = jnp.where(conf > _CONF, conf, -1.0)

    x1 = cx - w * 0.5
    y1 = cy - h * 0.5
    x2 = cx + w * 0.5
    y2 = cy + h * 0.5
    x1_ref[:, :] = x1
    y1_ref[:, :] = y1
    x2_ref[:, :] = x2
    y2_ref[:, :] = y2
    area_ref[:, :] = (x2 - x1) * (y2 - y1)
    pit_ref[:, :] = fields_ref[6]
    yaw_ref[:, :] = fields_ref[7]
    rol_ref[:, :] = fields_ref[8]

    s_int = jax.lax.bitcast_convert_type(s0, jnp.int32)
    kf = jnp.float32(_K)

    pos_cnt = jnp.sum(jnp.where(s_int >= 0, 1.0, 0.0))

    def bis_body(_, lh):
        lo, hi = lh
        mid = lo + (hi - lo + 1) // 2
        cnt = jnp.sum(jnp.where(s_int >= mid, 1.0, 0.0))
        ok = cnt >= kf
        return (jnp.where(ok, mid, lo), jnp.where(ok, hi, mid - 1))

    vpos, _ = jax.lax.fori_loop(
        0, 31, bis_body, (jnp.int32(0), jnp.int32(_BITS_ONE)))
    v = jnp.where(pos_cnt >= kf, vpos, jnp.int32(_BITS_NEG1))

    idx2d = (jax.lax.broadcasted_iota(jnp.int32, (_ROWS, _COLS), 0) * _COLS
             + jax.lax.broadcasted_iota(jnp.int32, (_ROWS, _COLS), 1))

    gt = s_int > v
    eq = s_int == v
    c_gt = jnp.sum(jnp.where(gt, 1.0, 0.0))

    def jbody(_, lh):
        lo, hi = lh
        mid = (lo + hi) // 2
        cnt = c_gt + jnp.sum(jnp.where(eq & (idx2d <= mid), 1.0, 0.0))
        ok = cnt >= kf
        return (jnp.where(ok, lo, mid + 1), jnp.where(ok, mid, hi))

    jcut, _ = jax.lax.fori_loop(
        0, 15, jbody, (jnp.int32(0), jnp.int32(_NPAD - 1)))

    keep = gt | (eq & (idx2d <= jcut))
    s_start = jnp.where(keep, s0, -1.0)

    oh_iota = (jax.lax.broadcasted_iota(jnp.int32, (8, _COLS), 0) * _COLS
               + jax.lax.broadcasted_iota(jnp.int32, (8, _COLS), 1))
    lane = jax.lax.broadcasted_iota(jnp.int32, (1, _COLS), 1)

    def step(t, carry):
        s, ax1, ay1, ax2, ay2, acf, apt, ayw, arl = carry
        m = jnp.max(s)
        valid = jnp.where(m > 0.0, 1.0, 0.0)
        bidx = jnp.min(jnp.where(s == m, idx2d, jnp.int32(_NPAD)))
        r = bidx // _COLS
        c = bidx % _COLS
        lm = jnp.where(lane == c, 1.0, 0.0)

        def ext(ref):
            return jnp.sum(ref[pl.ds(r, 1), :] * lm)

        bx1 = ext(x1_ref)
        by1 = ext(y1_ref)
        bx2 = ext(x2_ref)
        by2 = ext(y2_ref)
        bpt = ext(pit_ref)
        byw = ext(yaw_ref)
        brl = ext(rol_ref)
        ba = (bx2 - bx1) * (by2 - by1)

        ix1 = jnp.maximum(bx1, x1_ref[:, :])
        iy1 = jnp.maximum(by1, y1_ref[:, :])
        ix2 = jnp.minimum(bx2, x2_ref[:, :])
        iy2 = jnp.minimum(by2, y2_ref[:, :])
        inter = (jnp.clip(ix2 - ix1, 0.0) * jnp.clip(iy2 - iy1, 0.0))
        iou = inter / (ba + area_ref[:, :] - inter + 1e-9)
        supp = (iou > _IOU) | (idx2d == bidx)
        s = jnp.where(supp, -1.0, s)

        oh = jnp.where(oh_iota == t, valid, 0.0)
        return (s,
                ax1 + oh * bx1, ay1 + oh * by1,
                ax2 + oh * bx2, ay2 + oh * by2,
                acf + oh * m,
                apt + oh * bpt, ayw + oh * byw, arl + oh * brl)

    z = jnp.zeros((8, _COLS), jnp.float32)
    carry = jax.lax.fori_loop(
        0, _MAXDET, step,
        (s_start, z, z, z, z, z, z, z, z))
    _, ax1, ay1, ax2, ay2, acf, apt, ayw, arl = carry

    out_ref[0, :, :] = ax1
    out_ref[1, :, :] = ay1
    out_ref[2, :, :] = ax2
    out_ref[3, :, :] = ay2
    out_ref[4, :, :] = acf
    out_ref[5, :, :] = z
    out_ref[6, :, :] = apt
    out_ref[7, :, :] = ayw
    out_ref[8, :, :] = arl


def _nms_call(fields):
    scr = pltpu.VMEM((_ROWS, _COLS), jnp.float32)
    return pl.pallas_call(
        _nms_body,
        out_shape=jax.ShapeDtypeStruct((9, 8, _COLS), jnp.float32),
        scratch_shapes=[scr] * 8,
    )(fields)


def kernel(predictions):
    p = predictions[0]
    pt = jnp.pad(p.T, ((0, 0), (0, _NPAD - _N)))
    fields = pt.reshape(9, _ROWS, _COLS)
    out = _nms_call(fields)
    return out.reshape(9, 8 * _COLS)[:, :_MAXDET].T


# speedup vs baseline: 21.2125x; 1.1872x over previous
import jax
import jax.numpy as jnp
from jax.experimental import pallas as pl
from jax.experimental.pallas import tpu as pltpu

_CONF = 0.7
_IOU = 0.45
_MAXDET = 300
_K = 1000
_N = 20000
_ROWS = 160
_COLS = 128
_NPAD = _ROWS * _COLS

_BITS_NEG1 = -1082130432
_BITS_ONE = 1065353216


def _nms_body(fields_ref, out_ref, x1_ref, y1_ref, x2_ref, y2_ref,
              area_ref, pit_ref, yaw_ref, rol_ref, sm_ref, sem):
    copy = pltpu.make_async_copy(fields_ref, sm_ref, sem)
    copy.start()
    cx = fields_ref[0]
    cy = fields_ref[1]
    w = fields_ref[2]
    h = fields_ref[3]
    conf = fields_ref[4] * fields_ref[5]
    s0 = jnp.where(conf > _CONF, conf, -1.0)

    x1 = cx - w * 0.5
    y1 = cy - h * 0.5
    x2 = cx + w * 0.5
    y2 = cy + h * 0.5
    x1_ref[:, :] = x1
    y1_ref[:, :] = y1
    x2_ref[:, :] = x2
    y2_ref[:, :] = y2
    area_ref[:, :] = (x2 - x1) * (y2 - y1)
    pit_ref[:, :] = fields_ref[6]
    yaw_ref[:, :] = fields_ref[7]
    rol_ref[:, :] = fields_ref[8]

    s_int = jax.lax.bitcast_convert_type(s0, jnp.int32)
    kf = jnp.float32(_K)

    pos_cnt = jnp.sum(jnp.where(s_int >= 0, 1.0, 0.0))

    def bis_body(_, lh):
        lo, hi = lh
        mid = lo + (hi - lo + 1) // 2
        cnt = jnp.sum(jnp.where(s_int >= mid, 1.0, 0.0))
        ok = cnt >= kf
        return (jnp.where(ok, mid, lo), jnp.where(ok, hi, mid - 1))

    vpos, _ = jax.lax.fori_loop(
        0, 31, bis_body, (jnp.int32(0), jnp.int32(_BITS_ONE)))
    v = jnp.where(pos_cnt >= kf, vpos, jnp.int32(_BITS_NEG1))

    idx2d = (jax.lax.broadcasted_iota(jnp.int32, (_ROWS, _COLS), 0) * _COLS
             + jax.lax.broadcasted_iota(jnp.int32, (_ROWS, _COLS), 1))

    gt = s_int > v
    eq = s_int == v
    c_gt = jnp.sum(jnp.where(gt, 1.0, 0.0))

    def jbody(_, lh):
        lo, hi = lh
        mid = (lo + hi) // 2
        cnt = c_gt + jnp.sum(jnp.where(eq & (idx2d <= mid), 1.0, 0.0))
        ok = cnt >= kf
        return (jnp.where(ok, lo, mid + 1), jnp.where(ok, mid, hi))

    jcut, _ = jax.lax.fori_loop(
        0, 15, jbody, (jnp.int32(0), jnp.int32(_NPAD - 1)))

    keep = gt | (eq & (idx2d <= jcut))
    s_start = jnp.where(keep, s0, -1.0)

    oh_iota = (jax.lax.broadcasted_iota(jnp.int32, (8, _COLS), 0) * _COLS
               + jax.lax.broadcasted_iota(jnp.int32, (8, _COLS), 1))
    copy.wait()

    def step(t, carry):
        s, ax1, ay1, ax2, ay2, acf, apt, ayw, arl = carry
        m = jnp.max(s)
        valid = jnp.where(m > 0.0, 1.0, 0.0)
        bidx = jnp.min(jnp.where(s == m, idx2d, jnp.int32(_NPAD)))
        r = bidx // _COLS
        c = bidx % _COLS
        bcx = sm_ref[0, r, c]
        bcy = sm_ref[1, r, c]
        bw = sm_ref[2, r, c]
        bh = sm_ref[3, r, c]
        bpt = sm_ref[6, r, c]
        byw = sm_ref[7, r, c]
        brl = sm_ref[8, r, c]
        bx1 = bcx - bw * 0.5
        by1 = bcy - bh * 0.5
        bx2 = bcx + bw * 0.5
        by2 = bcy + bh * 0.5
        ba = (bx2 - bx1) * (by2 - by1)

        ix1 = jnp.maximum(bx1, x1_ref[:, :])
        iy1 = jnp.maximum(by1, y1_ref[:, :])
        ix2 = jnp.minimum(bx2, x2_ref[:, :])
        iy2 = jnp.minimum(by2, y2_ref[:, :])
        inter = (jnp.clip(ix2 - ix1, 0.0) * jnp.clip(iy2 - iy1, 0.0))
        iou = inter / (ba + area_ref[:, :] - inter + 1e-9)
        supp = (iou > _IOU) | (idx2d == bidx)
        s = jnp.where(supp, -1.0, s)

        oh = jnp.where(oh_iota == t, valid, 0.0)
        return (s,
                ax1 + oh * bx1, ay1 + oh * by1,
                ax2 + oh * bx2, ay2 + oh * by2,
                acf + oh * m,
                apt + oh * bpt, ayw + oh * byw, arl + oh * brl)

    z = jnp.zeros((8, _COLS), jnp.float32)
    carry = jax.lax.fori_loop(
        0, _MAXDET, step,
        (s_start, z, z, z, z, z, z, z, z))
    _, ax1, ay1, ax2, ay2, acf, apt, ayw, arl = carry

    out_ref[0, :, :] = ax1
    out_ref[1, :, :] = ay1
    out_ref[2, :, :] = ax2
    out_ref[3, :, :] = ay2
    out_ref[4, :, :] = acf
    out_ref[5, :, :] = z
    out_ref[6, :, :] = apt
    out_ref[7, :, :] = ayw
    out_ref[8, :, :] = arl


def _nms_call(fields):
    scr = pltpu.VMEM((_ROWS, _COLS), jnp.float32)
    return pl.pallas_call(
        _nms_body,
        out_shape=jax.ShapeDtypeStruct((9, 8, _COLS), jnp.float32),
        scratch_shapes=[scr] * 8 + [pltpu.SMEM((9, _ROWS, _COLS), jnp.float32),
                                    pltpu.SemaphoreType.DMA],
    )(fields)


def kernel(predictions):
    p = predictions[0]
    pt = jnp.pad(p.T, ((0, 0), (0, _NPAD - _N)))
    fields = pt.reshape(9, _ROWS, _COLS)
    out = _nms_call(fields)
    return out.reshape(9, 8 * _COLS)[:, :_MAXDET].T
